# baseline (speedup 1.0000x reference)
import functools
import math

import jax
import jax.numpy as jnp
from jax import lax
from jax.experimental import pallas as pl
from jax.experimental.pallas import tpu as pltpu
from jax.experimental.pallas import tpu_sc as plsc

N_EMB = 100000
EMB_DIM = 64
SIGMA1 = 1.0
SIGMA2 = 0.1
PI_MIX = 0.5

_BLKN = 4000
_GRID = N_EMB // _BLKN

_NC, _NS = 2, 16
_NW = _NC * _NS
_CH = 128
_B_TOTAL = 4096 * 50
_J = _B_TOTAL // (_NW * _CH)


def _dense_body(mu_ref, rho_ref, eps_ref, w_ref, sums_ref, acc_ref):
    step = pl.program_id(0)

    @pl.when(step == 0)
    def _init():
        acc_ref[0] = jnp.float32(0)
        acc_ref[1] = jnp.float32(0)
        acc_ref[2] = jnp.float32(0)

    rho = rho_ref[...]
    eps = eps_ref[...]
    sigma = jax.nn.softplus(rho) + 1e-5
    w = mu_ref[...] + eps * sigma
    w_ref[:, 0:EMB_DIM] = w
    acc_ref[0] = acc_ref[0] + jnp.sum(jnp.log(sigma))
    acc_ref[1] = acc_ref[1] + jnp.sum(eps * eps)
    acc_ref[2] = acc_ref[2] + jnp.sum(w * w)

    @pl.when(step == _GRID - 1)
    def _fin():
        sums_ref[0] = acc_ref[0]
        sums_ref[1] = acc_ref[1]
        sums_ref[2] = acc_ref[2]


_dense = pl.pallas_call(
    _dense_body,
    grid=(_GRID,),
    in_specs=[pl.BlockSpec((_BLKN, EMB_DIM), lambda i: (i, 0))] * 3,
    out_specs=[
        pl.BlockSpec((_BLKN, 128), lambda i: (i, 0)),
        pl.BlockSpec(memory_space=pltpu.SMEM),
    ],
    out_shape=[
        jax.ShapeDtypeStruct((N_EMB, 128), jnp.float32),
        jax.ShapeDtypeStruct((3,), jnp.float32),
    ],
    scratch_shapes=[pltpu.SMEM((3,), jnp.float32)],
)


@functools.cache
def _make_sc_gather():
    mesh = plsc.VectorSubcoreMesh(
        core_axis_name="c", subcore_axis_name="s", num_cores=_NC, num_subcores=_NS
    )

    @functools.partial(
        pl.kernel,
        out_type=jax.ShapeDtypeStruct((_B_TOTAL, 128), jnp.float32),
        mesh=mesh,
        scratch_types=[
            pltpu.VMEM((_J, _CH), jnp.int32),
            pltpu.VMEM((_CH, 128), jnp.float32),
            pltpu.SemaphoreType.DMA,
        ],
        compiler_params=pltpu.CompilerParams(use_tc_tiling_on_sc=False),
    )
    def _sc_gather(table_hbm, idx_hbm, out_hbm, idx_v, rows_v, sem):
        wid = lax.axis_index("s") * _NC + lax.axis_index("c")
        base = wid * (_J * _CH)
        pltpu.sync_copy(idx_hbm.at[wid], idx_v)

        def body(j, carry):
            pltpu.async_copy(table_hbm.at[idx_v.at[j]], rows_v, sem).wait()
            pltpu.sync_copy(rows_v, out_hbm.at[pl.ds(base + j * _CH, _CH)])
            return carry

        lax.fori_loop(0, _J, body, 0)

    return _sc_gather


def kernel(input, mu, rho, eps):
    w2, sums = _dense(mu, rho, eps)

    idx3 = input.astype(jnp.int32).reshape(_NW, _J, _CH)
    flat = _make_sc_gather()(w2, idx3)
    after_embed = flat[:, :EMB_DIM].reshape(input.shape[0], input.shape[1], EMB_DIM)

    s_logsig, s_eps2, s_w2 = sums[0], sums[1], sums[2]
    n = float(N_EMB * EMB_DIM)
    c = 0.5 * math.log(2.0 * math.pi)
    log_posterior = -s_logsig - n * c - 0.5 * s_eps2
    mix1 = (
        -n * math.log(SIGMA1) - n * c - 0.5 * s_w2 / (SIGMA1 * SIGMA1)
        + math.log(PI_MIX)
    )
    mix2 = (
        -n * math.log(SIGMA2) - n * c - 0.5 * s_w2 / (SIGMA2 * SIGMA2)
        + math.log(1.0 - PI_MIX)
    )
    log_prior = jnp.logaddexp(mix1, mix2)
    kl = log_posterior - log_prior
    return (after_embed, kl)


# speedup vs baseline: 1.9209x; 1.9209x over previous
import functools
import math

import jax
import jax.numpy as jnp
from jax import lax
from jax.experimental import pallas as pl
from jax.experimental.pallas import tpu as pltpu
from jax.experimental.pallas import tpu_sc as plsc

N_EMB = 100000
EMB_DIM = 64
SIGMA1 = 1.0
SIGMA2 = 0.1
PI_MIX = 0.5

_BLKL = 6400
_GRIDD = -(-N_EMB // _BLKL)

_NC, _NS = 2, 16
_NW = _NC * _NS
_CH = 128
_B = 4096
_S = 50
_B_TOTAL = _B * _S
_J = _B_TOTAL // (_NW * _CH)


def _dense_body(mu_ref, rho_ref, eps_ref, w_ref, sums_ref, acc_ref):
    step = pl.program_id(0)

    @pl.when(step == 0)
    def _init():
        acc_ref[0] = jnp.float32(0)
        acc_ref[1] = jnp.float32(0)
        acc_ref[2] = jnp.float32(0)

    rho = rho_ref[...]
    eps = eps_ref[...]
    sigma = jax.nn.softplus(rho) + 1e-5
    w = mu_ref[...] + eps * sigma

    eye = jnp.eye(EMB_DIM, dtype=jnp.float32)
    w_rows = jax.lax.dot_general(
        w, eye, (((0,), (0,)), ((), ())), preferred_element_type=jnp.float32
    )
    w_ref[:, 0:EMB_DIM] = w_rows

    valid = N_EMB - step * _BLKL
    mask = lax.broadcasted_iota(jnp.int32, (EMB_DIM, _BLKL), 1) < valid
    zero = jnp.float32(0)
    acc_ref[0] = acc_ref[0] + jnp.sum(jnp.where(mask, jnp.log(sigma), zero))
    acc_ref[1] = acc_ref[1] + jnp.sum(jnp.where(mask, eps * eps, zero))
    acc_ref[2] = acc_ref[2] + jnp.sum(jnp.where(mask, w * w, zero))

    @pl.when(step == _GRIDD - 1)
    def _fin():
        sums_ref[0] = acc_ref[0]
        sums_ref[1] = acc_ref[1]
        sums_ref[2] = acc_ref[2]


_dense = pl.pallas_call(
    _dense_body,
    grid=(_GRIDD,),
    in_specs=[pl.BlockSpec((EMB_DIM, _BLKL), lambda i: (0, i))] * 3,
    out_specs=[
        pl.BlockSpec((_BLKL, 128), lambda i: (i, 0)),
        pl.BlockSpec(memory_space=pltpu.SMEM),
    ],
    out_shape=[
        jax.ShapeDtypeStruct((N_EMB, 128), jnp.float32),
        jax.ShapeDtypeStruct((3,), jnp.float32),
    ],
    scratch_shapes=[pltpu.SMEM((3,), jnp.float32)],
)


@functools.cache
def _make_sc_gather():
    mesh = plsc.VectorSubcoreMesh(
        core_axis_name="c", subcore_axis_name="s", num_cores=_NC, num_subcores=_NS
    )

    @functools.partial(
        pl.kernel,
        out_type=jax.ShapeDtypeStruct((_B_TOTAL, 128), jnp.float32),
        mesh=mesh,
        scratch_types=[
            pltpu.VMEM((_J, _CH), jnp.int32),
            pltpu.VMEM((_CH, 128), jnp.float32),
            pltpu.SemaphoreType.DMA,
        ],
        compiler_params=pltpu.CompilerParams(use_tc_tiling_on_sc=False),
    )
    def _sc_gather(table_hbm, idx_hbm, out_hbm, idx_v, rows_v, sem):
        wid = lax.axis_index("s") * _NC + lax.axis_index("c")
        base = wid * (_J * _CH)
        pltpu.sync_copy(idx_hbm.at[wid], idx_v)

        def body(j, carry):
            pltpu.async_copy(table_hbm.at[idx_v.at[j]], rows_v, sem).wait()
            pltpu.sync_copy(rows_v, out_hbm.at[pl.ds(base + j * _CH, _CH)])
            return carry

        lax.fori_loop(0, _J, body, 0)

    return _sc_gather


def _trans_body(x_ref, o_ref):
    o_ref[...] = jnp.swapaxes(x_ref[:, 0:EMB_DIM], 0, 1)


_trans = pl.pallas_call(
    _trans_body,
    grid=(_S,),
    in_specs=[pl.BlockSpec((_B, 128), lambda i: (i, 0))],
    out_specs=pl.BlockSpec((EMB_DIM, _B), lambda i: (i, 0)),
    out_shape=jax.ShapeDtypeStruct((_S * EMB_DIM, _B), jnp.float32),
)


def kernel(input, mu, rho, eps):
    w2, sums = _dense(mu.T, rho.T, eps.T)

    idx3 = input.T.astype(jnp.int32).reshape(_NW, _J, _CH)
    flat = _make_sc_gather()(w2, idx3)

    out_t = _trans(flat)
    after_embed = out_t.reshape(_S, EMB_DIM, _B).transpose(2, 0, 1)

    s_logsig, s_eps2, s_w2 = sums[0], sums[1], sums[2]
    n = float(N_EMB * EMB_DIM)
    c = 0.5 * math.log(2.0 * math.pi)
    log_posterior = -s_logsig - n * c - 0.5 * s_eps2
    mix1 = (
        -n * math.log(SIGMA1) - n * c - 0.5 * s_w2 / (SIGMA1 * SIGMA1)
        + math.log(PI_MIX)
    )
    mix2 = (
        -n * math.log(SIGMA2) - n * c - 0.5 * s_w2 / (SIGMA2 * SIGMA2)
        + math.log(1.0 - PI_MIX)
    )
    log_prior = jnp.logaddexp(mix1, mix2)
    kl = log_posterior - log_prior
    return (after_embed, kl)


# speedup vs baseline: 2.3340x; 1.2150x over previous
import functools
import math

import jax
import jax.numpy as jnp
from jax import lax
from jax.experimental import pallas as pl
from jax.experimental.pallas import tpu as pltpu
from jax.experimental.pallas import tpu_sc as plsc

N_EMB = 100000
EMB_DIM = 64
SIGMA1 = 1.0
SIGMA2 = 0.1
PI_MIX = 0.5

_BLKL = 6400
_GRIDD = -(-N_EMB // _BLKL)

_NC, _NS = 2, 16
_NW = _NC * _NS
_CH = 128
_B = 4096
_S = 50
_B_TOTAL = _B * _S
_J = _B_TOTAL // (_NW * _CH)


def _dense_body(mu_ref, rho_ref, eps_ref, w_ref, sums_ref, acc_ref):
    step = pl.program_id(0)

    @pl.when(step == 0)
    def _init():
        acc_ref[0] = jnp.float32(0)
        acc_ref[1] = jnp.float32(0)
        acc_ref[2] = jnp.float32(0)

    rho = rho_ref[...]
    eps = eps_ref[...]
    sigma = jax.nn.softplus(rho) + 1e-5
    w = mu_ref[...] + eps * sigma

    eye = jnp.eye(EMB_DIM, dtype=jnp.float32)
    w_rows = jax.lax.dot_general(
        w, eye, (((0,), (0,)), ((), ())), preferred_element_type=jnp.float32
    )
    w_ref[:, 0:EMB_DIM] = w_rows

    valid = N_EMB - step * _BLKL
    mask = lax.broadcasted_iota(jnp.int32, (EMB_DIM, _BLKL), 1) < valid
    zero = jnp.float32(0)
    acc_ref[0] = acc_ref[0] + jnp.sum(jnp.where(mask, jnp.log(sigma), zero))
    acc_ref[1] = acc_ref[1] + jnp.sum(jnp.where(mask, eps * eps, zero))
    acc_ref[2] = acc_ref[2] + jnp.sum(jnp.where(mask, w * w, zero))

    @pl.when(step == _GRIDD - 1)
    def _fin():
        sums_ref[0] = acc_ref[0]
        sums_ref[1] = acc_ref[1]
        sums_ref[2] = acc_ref[2]


_dense = pl.pallas_call(
    _dense_body,
    grid=(_GRIDD,),
    in_specs=[pl.BlockSpec((EMB_DIM, _BLKL), lambda i: (0, i))] * 3,
    out_specs=[
        pl.BlockSpec((_BLKL, 128), lambda i: (i, 0)),
        pl.BlockSpec(memory_space=pltpu.SMEM),
    ],
    out_shape=[
        jax.ShapeDtypeStruct((N_EMB, 128), jnp.float32),
        jax.ShapeDtypeStruct((3,), jnp.float32),
    ],
    scratch_shapes=[pltpu.SMEM((3,), jnp.float32)],
)


@functools.cache
def _make_sc_gather(j_count):
    mesh = plsc.VectorSubcoreMesh(
        core_axis_name="c", subcore_axis_name="s", num_cores=_NC, num_subcores=_NS
    )

    @functools.partial(
        pl.kernel,
        out_type=jax.ShapeDtypeStruct((_NW * j_count * _CH, 128), jnp.float32),
        mesh=mesh,
        scratch_types=[
            pltpu.VMEM((j_count, _CH), jnp.int32),
            pltpu.VMEM((_CH, 128), jnp.float32),
            pltpu.VMEM((_CH, 128), jnp.float32),
            pltpu.SemaphoreType.DMA,
        ],
        compiler_params=pltpu.CompilerParams(use_tc_tiling_on_sc=False),
    )
    def _sc_gather(table_hbm, idx_hbm, out_hbm, idx_v, rv0, rv1, sem):
        wid = lax.axis_index("s") * _NC + lax.axis_index("c")
        base = wid * (j_count * _CH)
        pltpu.sync_copy(idx_hbm.at[wid], idx_v)

        pltpu.async_copy(table_hbm.at[idx_v.at[0]], rv0, sem)

        def body(g, carry):
            j0 = g * 2
            j1 = j0 + 1
            pltpu.async_copy(table_hbm.at[idx_v.at[j1]], rv1, sem)
            pltpu.make_async_copy(table_hbm.at[idx_v.at[j0]], rv0, sem).wait()
            pltpu.sync_copy(rv0, out_hbm.at[pl.ds(base + j0 * _CH, _CH)])

            @pl.when(j0 + 2 < j_count)
            def _fire():
                pltpu.async_copy(table_hbm.at[idx_v.at[j0 + 2]], rv0, sem)

            pltpu.make_async_copy(table_hbm.at[idx_v.at[j1]], rv1, sem).wait()
            pltpu.sync_copy(rv1, out_hbm.at[pl.ds(base + j1 * _CH, _CH)])
            return carry

        lax.fori_loop(0, j_count // 2, body, 0)

        if j_count % 2:
            j = j_count - 1
            pltpu.make_async_copy(table_hbm.at[idx_v.at[j]], rv0, sem).wait()
            pltpu.sync_copy(rv0, out_hbm.at[pl.ds(base + j * _CH, _CH)])

    return _sc_gather


_SH = _S // 2
_JH = _J // 2


def _trans_body(x_ref, o_ref):
    o_ref[...] = jnp.swapaxes(x_ref[:, 0:EMB_DIM], 0, 1)


def _trans_body2(prev_ref, x_ref, o_ref):
    o_ref[...] = jnp.swapaxes(x_ref[:, 0:EMB_DIM], 0, 1)


_trans0 = pl.pallas_call(
    _trans_body,
    grid=(_SH,),
    in_specs=[pl.BlockSpec((_B, 128), lambda i: (i, 0))],
    out_specs=pl.BlockSpec((EMB_DIM, _B), lambda i: (i, 0)),
    out_shape=jax.ShapeDtypeStruct((_S * EMB_DIM, _B), jnp.float32),
)

_trans1 = pl.pallas_call(
    _trans_body2,
    grid=(_SH,),
    in_specs=[
        pl.BlockSpec(memory_space=pl.ANY),
        pl.BlockSpec((_B, 128), lambda i: (i, 0)),
    ],
    out_specs=pl.BlockSpec((EMB_DIM, _B), lambda i: (_SH + i, 0)),
    out_shape=jax.ShapeDtypeStruct((_S * EMB_DIM, _B), jnp.float32),
    input_output_aliases={0: 0},
)


def kernel(input, mu, rho, eps):
    w2, sums = _dense(mu.T, rho.T, eps.T)

    idx_flat = input.T.astype(jnp.int32).reshape(-1)
    half = _B_TOTAL // 2
    gather = _make_sc_gather(_JH)
    flat0 = gather(w2, idx_flat[:half].reshape(_NW, _JH, _CH))
    flat1 = gather(w2, idx_flat[half:].reshape(_NW, _JH, _CH))

    out_half = _trans0(flat0)
    out_t = _trans1(out_half, flat1)
    after_embed = out_t.reshape(_S, EMB_DIM, _B).transpose(2, 0, 1)

    s_logsig, s_eps2, s_w2 = sums[0], sums[1], sums[2]
    n = float(N_EMB * EMB_DIM)
    c = 0.5 * math.log(2.0 * math.pi)
    log_posterior = -s_logsig - n * c - 0.5 * s_eps2
    mix1 = (
        -n * math.log(SIGMA1) - n * c - 0.5 * s_w2 / (SIGMA1 * SIGMA1)
        + math.log(PI_MIX)
    )
    mix2 = (
        -n * math.log(SIGMA2) - n * c - 0.5 * s_w2 / (SIGMA2 * SIGMA2)
        + math.log(1.0 - PI_MIX)
    )
    log_prior = jnp.logaddexp(mix1, mix2)
    kl = log_posterior - log_prior
    return (after_embed, kl)


# speedup vs baseline: 2.4845x; 1.0645x over previous
import functools
import math

import jax
import jax.numpy as jnp
from jax import lax
from jax.experimental import pallas as pl
from jax.experimental.pallas import tpu as pltpu
from jax.experimental.pallas import tpu_sc as plsc

N_EMB = 100000
EMB_DIM = 64
SIGMA1 = 1.0
SIGMA2 = 0.1
PI_MIX = 0.5

_BLKL = 6400
_GRIDD = -(-N_EMB // _BLKL)

_NC, _NS = 2, 16
_NW = _NC * _NS
_CH = 128
_B = 4096
_S = 50
_B_TOTAL = _B * _S
_J = _B_TOTAL // (_NW * _CH)


def _dense_body(mu_ref, rho_ref, eps_ref, w_ref, sums_ref, acc_ref):
    step = pl.program_id(0)

    @pl.when(step == 0)
    def _init():
        acc_ref[0] = jnp.float32(0)
        acc_ref[1] = jnp.float32(0)
        acc_ref[2] = jnp.float32(0)

    rho = rho_ref[...]
    eps = eps_ref[...]
    sigma = jax.nn.softplus(rho) + 1e-5
    w = mu_ref[...] + eps * sigma

    eye = jnp.eye(EMB_DIM, dtype=jnp.float32)
    w_rows = jax.lax.dot_general(
        w, eye, (((0,), (0,)), ((), ())), preferred_element_type=jnp.float32
    )
    w_ref[:, 0:EMB_DIM] = w_rows

    valid = N_EMB - step * _BLKL
    mask = lax.broadcasted_iota(jnp.int32, (EMB_DIM, _BLKL), 1) < valid
    zero = jnp.float32(0)
    acc_ref[0] = acc_ref[0] + jnp.sum(jnp.where(mask, jnp.log(sigma), zero))
    acc_ref[1] = acc_ref[1] + jnp.sum(jnp.where(mask, eps * eps, zero))
    acc_ref[2] = acc_ref[2] + jnp.sum(jnp.where(mask, w * w, zero))

    @pl.when(step == _GRIDD - 1)
    def _fin():
        sums_ref[0] = acc_ref[0]
        sums_ref[1] = acc_ref[1]
        sums_ref[2] = acc_ref[2]


_dense = pl.pallas_call(
    _dense_body,
    grid=(_GRIDD,),
    in_specs=[pl.BlockSpec((EMB_DIM, _BLKL), lambda i: (0, i))] * 3,
    out_specs=[
        pl.BlockSpec((_BLKL, 128), lambda i: (i, 0)),
        pl.BlockSpec(memory_space=pltpu.SMEM),
    ],
    out_shape=[
        jax.ShapeDtypeStruct((N_EMB, 128), jnp.float32),
        jax.ShapeDtypeStruct((3,), jnp.float32),
    ],
    scratch_shapes=[pltpu.SMEM((3,), jnp.float32)],
)


_T = (_B_TOTAL // 2) // (_NW * _CH)
_HB = _B // 2


@functools.cache
def _make_sc_gather():
    mesh = plsc.VectorSubcoreMesh(
        core_axis_name="c", subcore_axis_name="s", num_cores=_NC, num_subcores=_NS
    )

    @functools.partial(
        pl.kernel,
        out_type=jax.ShapeDtypeStruct((_B_TOTAL // 2, 128), jnp.float32),
        mesh=mesh,
        scratch_types=[
            pltpu.VMEM((_T, 2, _CH), jnp.int32),
            pltpu.VMEM((_CH, 128), jnp.float32),
            pltpu.VMEM((_CH, 128), jnp.float32),
            pltpu.VMEM((_CH, 128), jnp.float32),
            pltpu.VMEM((_CH, 128), jnp.float32),
            pltpu.SemaphoreType.DMA,
        ],
        compiler_params=pltpu.CompilerParams(use_tc_tiling_on_sc=False),
    )
    def _sc_gather(table_hbm, idx_hbm, out_hbm, idx_v, ga0, gb0, ga1, gb1, sem):
        wid = lax.axis_index("s") * _NC + lax.axis_index("c")
        t0 = wid * _T
        pltpu.sync_copy(idx_hbm.at[wid], idx_v)

        def row0(k):
            t = t0 + k
            return (t // 16) * _HB + (t % 16) * _CH

        def fire(k, ga, gb):
            pltpu.async_copy(table_hbm.at[idx_v.at[k, 0]], ga, sem)
            pltpu.async_copy(table_hbm.at[idx_v.at[k, 1]], gb, sem)

        def drain_scatter(k, ga, gb):
            pltpu.make_async_copy(table_hbm.at[idx_v.at[k, 0]], ga, sem).wait()
            pltpu.make_async_copy(table_hbm.at[idx_v.at[k, 1]], gb, sem).wait()
            r = row0(k)
            pltpu.sync_copy(
                ga.at[:, pl.ds(0, EMB_DIM)],
                out_hbm.at[pl.ds(r, _CH), pl.ds(0, EMB_DIM)],
            )
            pltpu.sync_copy(
                gb.at[:, pl.ds(0, EMB_DIM)],
                out_hbm.at[pl.ds(r, _CH), pl.ds(EMB_DIM, EMB_DIM)],
            )

        fire(0, ga0, gb0)

        def body(g, carry):
            k0 = g * 2
            k1 = k0 + 1
            fire(k1, ga1, gb1)
            drain_scatter(k0, ga0, gb0)

            @pl.when(k0 + 2 < _T)
            def _f():
                fire(k0 + 2, ga0, gb0)

            drain_scatter(k1, ga1, gb1)
            return carry

        lax.fori_loop(0, _T // 2, body, 0)

        if _T % 2:
            drain_scatter(_T - 1, ga0, gb0)

    return _sc_gather


def _trans_body(x_ref, o_ref):
    x = x_ref[...]
    ta = jnp.swapaxes(x[:, 0:EMB_DIM], 0, 1)
    tb = jnp.swapaxes(x[:, EMB_DIM:128], 0, 1)
    o_ref[...] = jnp.concatenate([ta, tb], axis=1)


_trans = pl.pallas_call(
    _trans_body,
    grid=(_S,),
    in_specs=[pl.BlockSpec((_HB, 128), lambda i: (i, 0))],
    out_specs=pl.BlockSpec((EMB_DIM, _B), lambda i: (i, 0)),
    out_shape=jax.ShapeDtypeStruct((_S * EMB_DIM, _B), jnp.float32),
)


def kernel(input, mu, rho, eps):
    w2, sums = _dense(mu.T, rho.T, eps.T)

    idx_t = input.T.astype(jnp.int32)
    idx5 = (
        idx_t.reshape(_S, 2, 16, _CH)
        .transpose(0, 2, 1, 3)
        .reshape(_NW, _T, 2, _CH)
    )
    packed = _make_sc_gather()(w2, idx5)

    out_t = _trans(packed)
    after_embed = out_t.reshape(_S, EMB_DIM, _B).transpose(2, 0, 1)

    s_logsig, s_eps2, s_w2 = sums[0], sums[1], sums[2]
    n = float(N_EMB * EMB_DIM)
    c = 0.5 * math.log(2.0 * math.pi)
    log_posterior = -s_logsig - n * c - 0.5 * s_eps2
    mix1 = (
        -n * math.log(SIGMA1) - n * c - 0.5 * s_w2 / (SIGMA1 * SIGMA1)
        + math.log(PI_MIX)
    )
    mix2 = (
        -n * math.log(SIGMA2) - n * c - 0.5 * s_w2 / (SIGMA2 * SIGMA2)
        + math.log(1.0 - PI_MIX)
    )
    log_prior = jnp.logaddexp(mix1, mix2)
    kl = log_posterior - log_prior
    return (after_embed, kl)


# speedup vs baseline: 2.5430x; 1.0235x over previous
import functools
import math

import jax
import jax.numpy as jnp
from jax import lax
from jax.experimental import pallas as pl
from jax.experimental.pallas import tpu as pltpu
from jax.experimental.pallas import tpu_sc as plsc

N_EMB = 100000
EMB_DIM = 64
SIGMA1 = 1.0
SIGMA2 = 0.1
PI_MIX = 0.5

_BLKL = 6400
_GRIDD = -(-N_EMB // _BLKL)

_NC, _NS = 2, 16
_NW = _NC * _NS
_CH = 128
_B = 4096
_S = 50
_B_TOTAL = _B * _S
_J = _B_TOTAL // (_NW * _CH)


def _dense_body(mu_ref, rho_ref, eps_ref, w_ref, sums_ref, acc_ref):
    step = pl.program_id(0)

    @pl.when(step == 0)
    def _init():
        acc_ref[0] = jnp.float32(0)
        acc_ref[1] = jnp.float32(0)
        acc_ref[2] = jnp.float32(0)

    rho = rho_ref[...]
    eps = eps_ref[...]
    sigma = jax.nn.softplus(rho) + 1e-5
    w = mu_ref[...] + eps * sigma

    eye = jnp.eye(EMB_DIM, dtype=jnp.float32)
    w_rows = jax.lax.dot_general(
        w, eye, (((0,), (0,)), ((), ())), preferred_element_type=jnp.float32
    )
    w_ref[:, 0:EMB_DIM] = w_rows

    valid = N_EMB - step * _BLKL
    mask = lax.broadcasted_iota(jnp.int32, (EMB_DIM, _BLKL), 1) < valid
    zero = jnp.float32(0)
    acc_ref[0] = acc_ref[0] + jnp.sum(jnp.where(mask, jnp.log(sigma), zero))
    acc_ref[1] = acc_ref[1] + jnp.sum(jnp.where(mask, eps * eps, zero))
    acc_ref[2] = acc_ref[2] + jnp.sum(jnp.where(mask, w * w, zero))

    @pl.when(step == _GRIDD - 1)
    def _fin():
        sums_ref[0] = acc_ref[0]
        sums_ref[1] = acc_ref[1]
        sums_ref[2] = acc_ref[2]


_dense = pl.pallas_call(
    _dense_body,
    grid=(_GRIDD,),
    in_specs=[pl.BlockSpec((EMB_DIM, _BLKL), lambda i: (0, i))] * 3,
    out_specs=[
        pl.BlockSpec((_BLKL, 128), lambda i: (i, 0)),
        pl.BlockSpec(memory_space=pltpu.SMEM),
    ],
    out_shape=[
        jax.ShapeDtypeStruct((N_EMB, 128), jnp.float32),
        jax.ShapeDtypeStruct((3,), jnp.float32),
    ],
    scratch_shapes=[pltpu.SMEM((3,), jnp.float32)],
)


_T = (_B_TOTAL // 2) // (_NW * _CH)
_HB = _B // 2


@functools.cache
def _make_sc_gather():
    mesh = plsc.VectorSubcoreMesh(
        core_axis_name="c", subcore_axis_name="s", num_cores=_NC, num_subcores=_NS
    )

    @functools.partial(
        pl.kernel,
        out_type=jax.ShapeDtypeStruct((_B_TOTAL // 2, 128), jnp.float32),
        mesh=mesh,
        scratch_types=[
            pltpu.VMEM((_T, 2, _CH), jnp.int32),
            pltpu.VMEM((_CH, 128), jnp.float32),
            pltpu.VMEM((_CH, 128), jnp.float32),
            pltpu.VMEM((_CH, 128), jnp.float32),
            pltpu.VMEM((_CH, 128), jnp.float32),
            pltpu.SemaphoreType.DMA,
        ],
        compiler_params=pltpu.CompilerParams(use_tc_tiling_on_sc=False),
    )
    def _sc_gather(table_hbm, idx_hbm, out_hbm, idx_v, ga0, gb0, ga1, gb1, sem):
        wid = lax.axis_index("s") * _NC + lax.axis_index("c")
        t0 = wid * _T
        pltpu.sync_copy(idx_hbm.at[wid], idx_v)

        def row0(k):
            t = t0 + k
            return (t // 16) * _HB + (t % 16) * _CH

        def fire(k, ga, gb):
            pltpu.async_copy(table_hbm.at[idx_v.at[k, 0]], ga, sem)
            pltpu.async_copy(table_hbm.at[idx_v.at[k, 1]], gb, sem)

        def drain_scatter(k, ga, gb):
            pltpu.make_async_copy(table_hbm.at[idx_v.at[k, 0]], ga, sem).wait()
            pltpu.make_async_copy(table_hbm.at[idx_v.at[k, 1]], gb, sem).wait()
            r = row0(k)
            pltpu.sync_copy(
                ga.at[:, pl.ds(0, EMB_DIM)],
                out_hbm.at[pl.ds(r, _CH), pl.ds(0, EMB_DIM)],
            )
            pltpu.sync_copy(
                gb.at[:, pl.ds(0, EMB_DIM)],
                out_hbm.at[pl.ds(r, _CH), pl.ds(EMB_DIM, EMB_DIM)],
            )

        fire(0, ga0, gb0)

        def body(g, carry):
            k0 = g * 2
            k1 = k0 + 1
            fire(k1, ga1, gb1)
            drain_scatter(k0, ga0, gb0)

            @pl.when(k0 + 2 < _T)
            def _f():
                fire(k0 + 2, ga0, gb0)

            drain_scatter(k1, ga1, gb1)
            return carry

        lax.fori_loop(0, _T // 2, body, 0)

        if _T % 2:
            drain_scatter(_T - 1, ga0, gb0)

    return _sc_gather


def _trans_body(x_ref, o_ref):
    x = x_ref[...]
    ta = jnp.swapaxes(x[:, 0:EMB_DIM], 0, 1)
    eye = jnp.eye(EMB_DIM, dtype=jnp.float32)
    tb = jax.lax.dot_general(
        eye, x[:, EMB_DIM:128], (((1,), (1,)), ((), ())),
        preferred_element_type=jnp.float32,
    )
    o_ref[...] = jnp.concatenate([ta, tb], axis=1)


_trans = pl.pallas_call(
    _trans_body,
    grid=(_S,),
    in_specs=[pl.BlockSpec((_HB, 128), lambda i: (i, 0))],
    out_specs=pl.BlockSpec((EMB_DIM, _B), lambda i: (i, 0)),
    out_shape=jax.ShapeDtypeStruct((_S * EMB_DIM, _B), jnp.float32),
)


def kernel(input, mu, rho, eps):
    w2, sums = _dense(mu.T, rho.T, eps.T)

    idx_t = input.T.astype(jnp.int32)
    idx5 = (
        idx_t.reshape(_S, 2, 16, _CH)
        .transpose(0, 2, 1, 3)
        .reshape(_NW, _T, 2, _CH)
    )
    packed = _make_sc_gather()(w2, idx5)

    out_t = _trans(packed)
    after_embed = out_t.reshape(_S, EMB_DIM, _B).transpose(2, 0, 1)

    s_logsig, s_eps2, s_w2 = sums[0], sums[1], sums[2]
    n = float(N_EMB * EMB_DIM)
    c = 0.5 * math.log(2.0 * math.pi)
    log_posterior = -s_logsig - n * c - 0.5 * s_eps2
    mix1 = (
        -n * math.log(SIGMA1) - n * c - 0.5 * s_w2 / (SIGMA1 * SIGMA1)
        + math.log(PI_MIX)
    )
    mix2 = (
        -n * math.log(SIGMA2) - n * c - 0.5 * s_w2 / (SIGMA2 * SIGMA2)
        + math.log(1.0 - PI_MIX)
    )
    log_prior = jnp.logaddexp(mix1, mix2)
    kl = log_posterior - log_prior
    return (after_embed, kl)


# speedup vs baseline: 2.7642x; 1.0870x over previous
import functools
import math

import jax
import jax.numpy as jnp
from jax import lax
from jax.experimental import pallas as pl
from jax.experimental.pallas import tpu as pltpu
from jax.experimental.pallas import tpu_sc as plsc

N_EMB = 100000
EMB_DIM = 64
SIGMA1 = 1.0
SIGMA2 = 0.1
PI_MIX = 0.5

_BLKL = 6400
_GRIDD = -(-N_EMB // _BLKL)

_NC, _NS = 2, 16
_NW = _NC * _NS
_CH = 128
_B = 4096
_S = 50
_B_TOTAL = _B * _S
_J = _B_TOTAL // (_NW * _CH)


def _dense_body(mu_ref, rho_ref, eps_ref, w_ref, sums_ref, acc_ref):
    step = pl.program_id(0)

    @pl.when(step == 0)
    def _init():
        acc_ref[0] = jnp.float32(0)
        acc_ref[1] = jnp.float32(0)
        acc_ref[2] = jnp.float32(0)

    rho = rho_ref[...]
    eps = eps_ref[...]
    sigma = jax.nn.softplus(rho) + 1e-5
    w = mu_ref[...] + eps * sigma

    eye = jnp.eye(EMB_DIM, dtype=jnp.float32)
    w_rows = jax.lax.dot_general(
        w, eye, (((0,), (0,)), ((), ())), preferred_element_type=jnp.float32
    )
    w_ref[:, 0:EMB_DIM] = w_rows

    valid = N_EMB - step * _BLKL
    mask = lax.broadcasted_iota(jnp.int32, (EMB_DIM, _BLKL), 1) < valid
    zero = jnp.float32(0)
    acc_ref[0] = acc_ref[0] + jnp.sum(jnp.where(mask, jnp.log(sigma), zero))
    acc_ref[1] = acc_ref[1] + jnp.sum(jnp.where(mask, eps * eps, zero))
    acc_ref[2] = acc_ref[2] + jnp.sum(jnp.where(mask, w * w, zero))

    @pl.when(step == _GRIDD - 1)
    def _fin():
        sums_ref[0] = acc_ref[0]
        sums_ref[1] = acc_ref[1]
        sums_ref[2] = acc_ref[2]


_dense = pl.pallas_call(
    _dense_body,
    grid=(_GRIDD,),
    in_specs=[pl.BlockSpec((EMB_DIM, _BLKL), lambda i: (0, i))] * 3,
    out_specs=[
        pl.BlockSpec((_BLKL, 128), lambda i: (i, 0)),
        pl.BlockSpec(memory_space=pltpu.SMEM),
    ],
    out_shape=[
        jax.ShapeDtypeStruct((N_EMB, 128), jnp.float32),
        jax.ShapeDtypeStruct((3,), jnp.float32),
    ],
    scratch_shapes=[pltpu.SMEM((3,), jnp.float32)],
)


_T = (_B_TOTAL // 2) // (_NW * _CH)
_HB = _B // 2


@functools.cache
def _make_sc_gather():
    mesh = plsc.VectorSubcoreMesh(
        core_axis_name="c", subcore_axis_name="s", num_cores=_NC, num_subcores=_NS
    )

    @functools.partial(
        pl.kernel,
        out_type=jax.ShapeDtypeStruct((_B_TOTAL // 2, 128), jnp.float32),
        mesh=mesh,
        scratch_types=[
            pltpu.VMEM((_T, 2, _CH), jnp.int32),
            pltpu.VMEM((_CH, 128), jnp.float32),
            pltpu.VMEM((_CH, 128), jnp.float32),
            pltpu.VMEM((_CH, 128), jnp.float32),
            pltpu.VMEM((_CH, 128), jnp.float32),
            pltpu.SemaphoreType.DMA,
        ],
        compiler_params=pltpu.CompilerParams(use_tc_tiling_on_sc=False),
    )
    def _sc_gather(table_hbm, idx_hbm, out_hbm, idx_v, ga0, gb0, ga1, gb1, sem):
        wid = lax.axis_index("s") * _NC + lax.axis_index("c")
        t0 = wid * _T
        pltpu.sync_copy(idx_hbm.at[wid], idx_v)

        def row0(k):
            t = t0 + k
            return (t // 16) * _HB + (t % 16) * _CH

        def fire(k, ga, gb):
            pltpu.async_copy(table_hbm.at[idx_v.at[k, 0]], ga, sem)
            pltpu.async_copy(table_hbm.at[idx_v.at[k, 1]], gb, sem)

        def drain_scatter(k, ga, gb):
            pltpu.make_async_copy(table_hbm.at[idx_v.at[k, 0]], ga, sem).wait()
            pltpu.make_async_copy(table_hbm.at[idx_v.at[k, 1]], gb, sem).wait()
            r = row0(k)
            pltpu.sync_copy(
                ga.at[:, pl.ds(0, EMB_DIM)],
                out_hbm.at[pl.ds(r, _CH), pl.ds(0, EMB_DIM)],
            )
            pltpu.sync_copy(
                gb.at[:, pl.ds(0, EMB_DIM)],
                out_hbm.at[pl.ds(r, _CH), pl.ds(EMB_DIM, EMB_DIM)],
            )

        fire(0, ga0, gb0)

        def body(g, carry):
            k0 = g * 2
            k1 = k0 + 1
            fire(k1, ga1, gb1)
            drain_scatter(k0, ga0, gb0)

            @pl.when(k0 + 2 < _T)
            def _f():
                fire(k0 + 2, ga0, gb0)

            drain_scatter(k1, ga1, gb1)
            return carry

        lax.fori_loop(0, _T // 2, body, 0)

        if _T % 2:
            drain_scatter(_T - 1, ga0, gb0)

    return _sc_gather


def _trans_body(x_ref, o_ref):
    eye = jnp.eye(EMB_DIM, dtype=jnp.float32)
    for h in range(2):
        x = x_ref[h * _HB:(h + 1) * _HB, :]
        ta = jnp.swapaxes(x[:, 0:EMB_DIM], 0, 1)
        tb = jax.lax.dot_general(
            eye, x[:, EMB_DIM:128], (((1,), (1,)), ((), ())),
            preferred_element_type=jnp.float32,
        )
        o_ref[h * EMB_DIM:(h + 1) * EMB_DIM, :] = jnp.concatenate(
            [ta, tb], axis=1
        )


_trans = pl.pallas_call(
    _trans_body,
    grid=(_S // 2,),
    in_specs=[pl.BlockSpec((2 * _HB, 128), lambda i: (i, 0))],
    out_specs=pl.BlockSpec((2 * EMB_DIM, _B), lambda i: (i, 0)),
    out_shape=jax.ShapeDtypeStruct((_S * EMB_DIM, _B), jnp.float32),
)


def kernel(input, mu, rho, eps):
    w2, sums = _dense(mu.T, rho.T, eps.T)

    idx_t = input.T.astype(jnp.int32)
    idx5 = (
        idx_t.reshape(_S, 2, 16, _CH)
        .transpose(0, 2, 1, 3)
        .reshape(_NW, _T, 2, _CH)
    )
    packed = _make_sc_gather()(w2, idx5)

    out_t = _trans(packed)
    after_embed = out_t.reshape(_S, EMB_DIM, _B).transpose(2, 0, 1)

    s_logsig, s_eps2, s_w2 = sums[0], sums[1], sums[2]
    n = float(N_EMB * EMB_DIM)
    c = 0.5 * math.log(2.0 * math.pi)
    log_posterior = -s_logsig - n * c - 0.5 * s_eps2
    mix1 = (
        -n * math.log(SIGMA1) - n * c - 0.5 * s_w2 / (SIGMA1 * SIGMA1)
        + math.log(PI_MIX)
    )
    mix2 = (
        -n * math.log(SIGMA2) - n * c - 0.5 * s_w2 / (SIGMA2 * SIGMA2)
        + math.log(1.0 - PI_MIX)
    )
    log_prior = jnp.logaddexp(mix1, mix2)
    kl = log_posterior - log_prior
    return (after_embed, kl)


# speedup vs baseline: 2.9173x; 1.0554x over previous
import functools
import math

import jax
import jax.numpy as jnp
from jax import lax
from jax.experimental import pallas as pl
from jax.experimental.pallas import tpu as pltpu
from jax.experimental.pallas import tpu_sc as plsc

N_EMB = 100000
EMB_DIM = 64
SIGMA1 = 1.0
SIGMA2 = 0.1
PI_MIX = 0.5

_BLKL = 6400
_GRIDD = -(-N_EMB // _BLKL)

_NC, _NS = 2, 16
_NW = _NC * _NS
_CH = 128
_B = 4096
_S = 50
_B_TOTAL = _B * _S
_J = _B_TOTAL // (_NW * _CH)


def _dense_body(mu_ref, rho_ref, eps_ref, w_ref, sums_ref, acc_ref):
    step = pl.program_id(0)

    @pl.when(step == 0)
    def _init():
        acc_ref[0] = jnp.float32(0)
        acc_ref[1] = jnp.float32(0)
        acc_ref[2] = jnp.float32(0)

    rho = rho_ref[...]
    eps = eps_ref[...]
    sigma = jax.nn.softplus(rho) + 1e-5
    w = mu_ref[...] + eps * sigma

    eye = jnp.eye(EMB_DIM, dtype=jnp.float32)
    w_rows = jax.lax.dot_general(
        w, eye, (((0,), (0,)), ((), ())), preferred_element_type=jnp.float32
    )
    w_ref[:, 0:EMB_DIM] = w_rows

    valid = N_EMB - step * _BLKL
    mask = lax.broadcasted_iota(jnp.int32, (EMB_DIM, _BLKL), 1) < valid
    zero = jnp.float32(0)
    acc_ref[0] = acc_ref[0] + jnp.sum(jnp.where(mask, jnp.log(sigma), zero))
    acc_ref[1] = acc_ref[1] + jnp.sum(jnp.where(mask, eps * eps, zero))
    acc_ref[2] = acc_ref[2] + jnp.sum(jnp.where(mask, w * w, zero))

    @pl.when(step == _GRIDD - 1)
    def _fin():
        sums_ref[0] = acc_ref[0]
        sums_ref[1] = acc_ref[1]
        sums_ref[2] = acc_ref[2]


_dense = pl.pallas_call(
    _dense_body,
    grid=(_GRIDD,),
    in_specs=[pl.BlockSpec((EMB_DIM, _BLKL), lambda i: (0, i))] * 3,
    out_specs=[
        pl.BlockSpec((_BLKL, 128), lambda i: (i, 0)),
        pl.BlockSpec(memory_space=pltpu.SMEM),
    ],
    out_shape=[
        jax.ShapeDtypeStruct((N_EMB, 128), jnp.float32),
        jax.ShapeDtypeStruct((3,), jnp.float32),
    ],
    scratch_shapes=[pltpu.SMEM((3,), jnp.float32)],
)


_T = (_B_TOTAL // 2) // (_NW * _CH)
_HB = _B // 2


@functools.cache
def _make_sc_gather():
    mesh = plsc.VectorSubcoreMesh(
        core_axis_name="c", subcore_axis_name="s", num_cores=_NC, num_subcores=_NS
    )

    @functools.partial(
        pl.kernel,
        out_type=jax.ShapeDtypeStruct((_B_TOTAL // 2, 128), jnp.float32),
        mesh=mesh,
        scratch_types=[
            pltpu.VMEM((_T, 2, _CH), jnp.int32),
            pltpu.VMEM((_CH, 128), jnp.float32),
            pltpu.VMEM((_CH, 128), jnp.float32),
            pltpu.VMEM((_CH, 128), jnp.float32),
            pltpu.VMEM((_CH, 128), jnp.float32),
            pltpu.SemaphoreType.DMA,
        ],
        compiler_params=pltpu.CompilerParams(use_tc_tiling_on_sc=False),
    )
    def _sc_gather(table_hbm, idx_hbm, out_hbm, idx_v, ga0, gb0, ga1, gb1, sem):
        wid = lax.axis_index("s") * _NC + lax.axis_index("c")
        t0 = wid * _T
        pltpu.sync_copy(idx_hbm.at[wid], idx_v)

        def row0(k):
            t = t0 + k
            return (t // 16) * _HB + (t % 16) * _CH

        def fire(k, ga, gb):
            pltpu.async_copy(table_hbm.at[idx_v.at[k, 0]], ga, sem)
            pltpu.async_copy(table_hbm.at[idx_v.at[k, 1]], gb, sem)

        def drain_scatter(k, ga, gb):
            pltpu.make_async_copy(table_hbm.at[idx_v.at[k, 0]], ga, sem).wait()
            pltpu.make_async_copy(table_hbm.at[idx_v.at[k, 1]], gb, sem).wait()
            r = row0(k)
            pltpu.sync_copy(
                ga.at[:, pl.ds(0, EMB_DIM)],
                out_hbm.at[pl.ds(r, _CH), pl.ds(0, EMB_DIM)],
            )
            pltpu.sync_copy(
                gb.at[:, pl.ds(0, EMB_DIM)],
                out_hbm.at[pl.ds(r, _CH), pl.ds(EMB_DIM, EMB_DIM)],
            )

        fire(0, ga0, gb0)

        def body(g, carry):
            k0 = g * 2
            k1 = k0 + 1
            fire(k1, ga1, gb1)
            drain_scatter(k0, ga0, gb0)

            @pl.when(k0 + 2 < _T)
            def _f():
                fire(k0 + 2, ga0, gb0)

            drain_scatter(k1, ga1, gb1)
            return carry

        lax.fori_loop(0, _T // 2, body, 0)

        if _T % 2:
            drain_scatter(_T - 1, ga0, gb0)

    return _sc_gather


_SPB = 5


def _trans_body(x_ref, o_ref):
    eye = jnp.eye(EMB_DIM, dtype=jnp.float32)
    for h in range(_SPB):
        x = x_ref[h * _HB:(h + 1) * _HB, :]
        ta = jnp.swapaxes(x[:, 0:EMB_DIM], 0, 1)
        tb = jax.lax.dot_general(
            eye, x[:, EMB_DIM:128], (((1,), (1,)), ((), ())),
            preferred_element_type=jnp.float32,
        )
        o_ref[h * EMB_DIM:(h + 1) * EMB_DIM, :] = jnp.concatenate(
            [ta, tb], axis=1
        )


_trans = pl.pallas_call(
    _trans_body,
    grid=(_S // _SPB,),
    in_specs=[pl.BlockSpec((_SPB * _HB, 128), lambda i: (i, 0))],
    out_specs=pl.BlockSpec((_SPB * EMB_DIM, _B), lambda i: (i, 0)),
    out_shape=jax.ShapeDtypeStruct((_S * EMB_DIM, _B), jnp.float32),
)


def kernel(input, mu, rho, eps):
    w2, sums = _dense(mu.T, rho.T, eps.T)

    idx_t = input.T.astype(jnp.int32)
    idx5 = (
        idx_t.reshape(_S, 2, 16, _CH)
        .transpose(0, 2, 1, 3)
        .reshape(_NW, _T, 2, _CH)
    )
    packed = _make_sc_gather()(w2, idx5)

    out_t = _trans(packed)
    after_embed = out_t.reshape(_S, EMB_DIM, _B).transpose(2, 0, 1)

    s_logsig, s_eps2, s_w2 = sums[0], sums[1], sums[2]
    n = float(N_EMB * EMB_DIM)
    c = 0.5 * math.log(2.0 * math.pi)
    log_posterior = -s_logsig - n * c - 0.5 * s_eps2
    mix1 = (
        -n * math.log(SIGMA1) - n * c - 0.5 * s_w2 / (SIGMA1 * SIGMA1)
        + math.log(PI_MIX)
    )
    mix2 = (
        -n * math.log(SIGMA2) - n * c - 0.5 * s_w2 / (SIGMA2 * SIGMA2)
        + math.log(1.0 - PI_MIX)
    )
    log_prior = jnp.logaddexp(mix1, mix2)
    kl = log_posterior - log_prior
    return (after_embed, kl)
